# speedup vs baseline: 1.2033x; 1.2033x over previous
import functools

import jax
import jax.numpy as jnp
import numpy as np
from jax.experimental import pallas as pl
from jax.experimental.pallas import tpu as pltpu

N = 4096
INPUT_DIM = 1280
H = 128
F = 128
L = 6
NG = 50
CUTOFF = 10.0
MAXNN = 32
NGRAPHS = 8
LOG2 = float(np.log(2.0))


def _ssp(v):
    return jax.nn.softplus(v) - LOG2


def _matmul_kern(x_ref, w_ref, b_ref, o_ref):
    o_ref[...] = jnp.dot(x_ref[...], w_ref[...],
                         preferred_element_type=jnp.float32) + b_ref[...]


def _input_proj(x, wt, b):
    BM = 512
    return pl.pallas_call(
        _matmul_kern,
        grid=(N // BM,),
        in_specs=[
            pl.BlockSpec((BM, INPUT_DIM), lambda i: (i, 0)),
            pl.BlockSpec((INPUT_DIM, H), lambda i: (0, 0)),
            pl.BlockSpec((1, H), lambda i: (0, 0)),
        ],
        out_specs=pl.BlockSpec((BM, H), lambda i: (i, 0)),
        out_shape=jax.ShapeDtypeStruct((N, H), jnp.float32),
    )(x, wt, b)


def kernel(x, pos, batch, input_proj_w, input_proj_b, mlp_w1, mlp_b1,
           mlp_w2, mlp_b2, cf_lin1_w, cf_lin2_w, cf_lin2_b, int_lin_w,
           int_lin_b, out1_w, out1_b, out2_w, out2_b):
    h = _input_proj(x, input_proj_w.T, input_proj_b[None, :])

    sq = jnp.sum(pos * pos, axis=1)
    d2 = jnp.maximum(sq[:, None] + sq[None, :] - 2.0 * (pos @ pos.T), 0.0)
    dist = jnp.sqrt(jnp.where(d2 > 0, d2, 1.0))
    same = batch[:, None] == batch[None, :]
    eye = jnp.eye(N, dtype=bool)
    mask = same & (~eye) & (dist < CUTOFF)
    md = jnp.where(mask, dist, jnp.inf)
    negv, nbr = jax.lax.top_k(-md, MAXNN)
    w = -negv
    valid = jnp.isfinite(w)
    w = jnp.where(valid, w, 0.0)
    src = nbr.reshape(-1)
    ew = w.reshape(-1)
    vm = valid.reshape(-1).astype(jnp.float32)

    offset = jnp.linspace(0.0, CUTOFF, NG)
    coeff = -0.5 / float(CUTOFF / (NG - 1)) ** 2
    edge_attr = jnp.exp(coeff * (ew[:, None] - offset[None, :]) ** 2)
    C = 0.5 * (jnp.cos(ew * jnp.pi / CUTOFF) + 1.0) * vm

    for l in range(L):
        Wf = _ssp(edge_attr @ mlp_w1[l].T + mlp_b1[l]) @ mlp_w2[l].T + mlp_b2[l]
        Wf = Wf * C[:, None]
        xs = h @ cf_lin1_w[l].T
        msg = xs[src] * Wf
        agg = msg.reshape(N, MAXNN, H).sum(axis=1)
        hc = _ssp(agg @ cf_lin2_w[l].T + cf_lin2_b[l])
        hc = hc @ int_lin_w[l].T + int_lin_b[l]
        h = h + hc

    h = _ssp(h @ out1_w.T + out1_b)
    h = h @ out2_w.T + out2_b
    return jax.ops.segment_sum(h, batch, num_segments=NGRAPHS)


# speedup vs baseline: 1.6663x; 1.3848x over previous
import functools

import jax
import jax.numpy as jnp
import numpy as np
from jax import lax
from jax.experimental import pallas as pl
from jax.experimental.pallas import tpu as pltpu
from jax.experimental.pallas import tpu_sc as plsc

N = 4096
INPUT_DIM = 1280
H = 128
F = 128
L = 6
NG = 50
CUTOFF = 10.0
MAXNN = 32
NGRAPHS = 8
LOG2 = float(np.log(2.0))

NW = 32
NODES_PW = N // NW
EDGES_PW = NODES_PW * MAXNN
ECHUNK = 128
NCHUNKS = EDGES_PW // ECHUNK
NBUF = 3
NVEC = H // 16


def _ssp(v):
    return jax.nn.softplus(v) - LOG2


def _matmul_kern(x_ref, w_ref, b_ref, o_ref):
    o_ref[...] = jnp.dot(x_ref[...], w_ref[...],
                         preferred_element_type=jnp.float32) + b_ref[...]


def _input_proj(x, wt, b):
    BM = 512
    return pl.pallas_call(
        _matmul_kern,
        grid=(N // BM,),
        in_specs=[
            pl.BlockSpec((BM, INPUT_DIM), lambda i: (i, 0)),
            pl.BlockSpec((INPUT_DIM, H), lambda i: (0, 0)),
            pl.BlockSpec((1, H), lambda i: (0, 0)),
        ],
        out_specs=pl.BlockSpec((BM, H), lambda i: (i, 0)),
        out_shape=jax.ShapeDtypeStruct((N, H), jnp.float32),
    )(x, wt, b)


def _sc_agg_body(nbr_hbm, xs_hbm, wf_hbm, out_hbm, idx_v,
                 g0, g1, g2, w0, w1, w2, acc,
                 sg0, sg1, sg2, sw0, sw1, sw2):
    gbufs = (g0, g1, g2)
    wbufs = (w0, w1, w2)
    gsems = (sg0, sg1, sg2)
    wsems = (sw0, sw1, sw2)
    wid = lax.axis_index("s") * 2 + lax.axis_index("c")
    ebase = wid * EDGES_PW

    pltpu.sync_copy(nbr_hbm.at[pl.ds(wid * NCHUNKS, NCHUNKS)], idx_v)

    def issue(c, b):
        @pl.when(c < NCHUNKS)
        def _():
            pltpu.async_copy(xs_hbm.at[idx_v.at[c]], gbufs[b], gsems[b])
            pltpu.async_copy(wf_hbm.at[pl.ds(ebase + c * ECHUNK, ECHUNK)],
                             wbufs[b], wsems[b])

    def wait(c, b):
        pltpu.make_async_copy(xs_hbm.at[idx_v.at[c]], gbufs[b],
                              gsems[b]).wait()
        pltpu.make_async_copy(wf_hbm.at[pl.ds(ebase + c * ECHUNK, ECHUNK)],
                              wbufs[b], wsems[b]).wait()

    def compute(c, b):
        gb, wb = gbufs[b], wbufs[b]
        for n4 in range(ECHUNK // MAXNN):
            def mac(j, carry):
                e = n4 * MAXNN + j
                new = tuple(
                    carry[d] + gb[e, pl.ds(16 * d, 16)] * wb[e, pl.ds(16 * d, 16)]
                    for d in range(NVEC))
                return new
            zeros = tuple(jnp.zeros((16,), jnp.float32) for _ in range(NVEC))
            accv = lax.fori_loop(0, MAXNN, mac, zeros)
            nl = c * (ECHUNK // MAXNN) + n4
            for d in range(NVEC):
                acc[nl, pl.ds(16 * d, 16)] = accv[d]

    for b in range(NBUF):
        issue(b, b)
    def round_body(r, _):
        for b in range(NBUF):
            c = r * NBUF + b
            @pl.when(c < NCHUNKS)
            def _():
                wait(c, b)
                compute(c, b)
                issue(c + NBUF, b)
        return 0
    lax.fori_loop(0, (NCHUNKS + NBUF - 1) // NBUF, round_body, 0)

    pltpu.sync_copy(acc, out_hbm.at[pl.ds(wid * NODES_PW, NODES_PW)])


@functools.partial(
    pl.kernel,
    out_type=jax.ShapeDtypeStruct((N, H), jnp.float32),
    mesh=plsc.VectorSubcoreMesh(core_axis_name="c", subcore_axis_name="s"),
    scratch_types=[
        pltpu.VMEM((NCHUNKS, ECHUNK), jnp.int32),
    ] + [pltpu.VMEM((ECHUNK, H), jnp.float32)] * 6
      + [pltpu.VMEM((NODES_PW, H), jnp.float32)]
      + [pltpu.SemaphoreType.DMA] * 6,
)
def _sc_aggregate(nbr_hbm, xs_hbm, wf_hbm, out_hbm, idx_v,
                  g0, g1, g2, w0, w1, w2, acc,
                  sg0, sg1, sg2, sw0, sw1, sw2):
    _sc_agg_body(nbr_hbm, xs_hbm, wf_hbm, out_hbm, idx_v,
                 g0, g1, g2, w0, w1, w2, acc,
                 sg0, sg1, sg2, sw0, sw1, sw2)


def kernel(x, pos, batch, input_proj_w, input_proj_b, mlp_w1, mlp_b1,
           mlp_w2, mlp_b2, cf_lin1_w, cf_lin2_w, cf_lin2_b, int_lin_w,
           int_lin_b, out1_w, out1_b, out2_w, out2_b):
    h = _input_proj(x, input_proj_w.T, input_proj_b[None, :])

    sq = jnp.sum(pos * pos, axis=1)
    d2 = jnp.maximum(sq[:, None] + sq[None, :] - 2.0 * (pos @ pos.T), 0.0)
    dist = jnp.sqrt(jnp.where(d2 > 0, d2, 1.0))
    same = batch[:, None] == batch[None, :]
    eye = jnp.eye(N, dtype=bool)
    mask = same & (~eye) & (dist < CUTOFF)
    md = jnp.where(mask, dist, jnp.inf)
    negv, nbr = jax.lax.top_k(-md, MAXNN)
    w = -negv
    valid = jnp.isfinite(w)
    w = jnp.where(valid, w, 0.0)
    ew = w.reshape(-1)
    vm = valid.reshape(-1).astype(jnp.float32)
    nbr_dma = nbr.reshape(N * MAXNN // ECHUNK, ECHUNK)

    offset = jnp.linspace(0.0, CUTOFF, NG)
    coeff = -0.5 / float(CUTOFF / (NG - 1)) ** 2
    edge_attr = jnp.exp(coeff * (ew[:, None] - offset[None, :]) ** 2)
    C = 0.5 * (jnp.cos(ew * jnp.pi / CUTOFF) + 1.0) * vm

    for l in range(L):
        Wf = _ssp(edge_attr @ mlp_w1[l].T + mlp_b1[l]) @ mlp_w2[l].T + mlp_b2[l]
        Wf = Wf * C[:, None]
        xs = h @ cf_lin1_w[l].T
        agg = _sc_aggregate(nbr_dma, xs, Wf)
        hc = _ssp(agg @ cf_lin2_w[l].T + cf_lin2_b[l])
        hc = hc @ int_lin_w[l].T + int_lin_b[l]
        h = h + hc

    h = _ssp(h @ out1_w.T + out1_b)
    h = h @ out2_w.T + out2_b
    return jax.ops.segment_sum(h, batch, num_segments=NGRAPHS)


# speedup vs baseline: 10.6242x; 6.3760x over previous
import functools

import jax
import jax.numpy as jnp
import numpy as np
from jax import lax
from jax.experimental import pallas as pl
from jax.experimental.pallas import tpu as pltpu
from jax.experimental.pallas import tpu_sc as plsc

N = 4096
INPUT_DIM = 1280
H = 128
F = 128
L = 6
NG = 50
CUTOFF = 10.0
MAXNN = 32
NGRAPHS = 8
LOG2 = float(np.log(2.0))

NW = 32
NODES_PW = N // NW
EDGES_PW = NODES_PW * MAXNN
ECHUNK = 128
NCHUNKS = EDGES_PW // ECHUNK
NBUF = 3
NVEC = H // 16


def _ssp(v):
    return jax.nn.softplus(v) - LOG2


def _matmul_kern(x_ref, w_ref, b_ref, o_ref):
    o_ref[...] = jnp.dot(x_ref[...], w_ref[...],
                         preferred_element_type=jnp.float32) + b_ref[...]


def _input_proj(x, wt, b):
    BM = 512
    return pl.pallas_call(
        _matmul_kern,
        grid=(N // BM,),
        in_specs=[
            pl.BlockSpec((BM, INPUT_DIM), lambda i: (i, 0)),
            pl.BlockSpec((INPUT_DIM, H), lambda i: (0, 0)),
            pl.BlockSpec((1, H), lambda i: (0, 0)),
        ],
        out_specs=pl.BlockSpec((BM, H), lambda i: (i, 0)),
        out_shape=jax.ShapeDtypeStruct((N, H), jnp.float32),
    )(x, wt, b)


def _sc_agg_body(nbr_hbm, xs_hbm, wf_hbm, out_hbm, idx_v,
                 g0, g1, g2, w0, w1, w2, acc,
                 sg0, sg1, sg2, sw0, sw1, sw2):
    gbufs = (g0, g1, g2)
    wbufs = (w0, w1, w2)
    gsems = (sg0, sg1, sg2)
    wsems = (sw0, sw1, sw2)
    wid = lax.axis_index("s") * 2 + lax.axis_index("c")
    ebase = wid * EDGES_PW

    pltpu.sync_copy(nbr_hbm.at[pl.ds(wid * NCHUNKS, NCHUNKS)], idx_v)

    def issue(c, b):
        @pl.when(c < NCHUNKS)
        def _():
            pltpu.async_copy(xs_hbm.at[idx_v.at[c]], gbufs[b], gsems[b])
            pltpu.async_copy(wf_hbm.at[pl.ds(ebase + c * ECHUNK, ECHUNK)],
                             wbufs[b], wsems[b])

    def wait(c, b):
        pltpu.make_async_copy(xs_hbm.at[idx_v.at[c]], gbufs[b],
                              gsems[b]).wait()
        pltpu.make_async_copy(wf_hbm.at[pl.ds(ebase + c * ECHUNK, ECHUNK)],
                              wbufs[b], wsems[b]).wait()

    def compute(c, b):
        gb, wb = gbufs[b], wbufs[b]
        for n4 in range(ECHUNK // MAXNN):
            def mac(j, carry):
                e = n4 * MAXNN + j
                new = tuple(
                    carry[d] + gb[e, pl.ds(16 * d, 16)] * wb[e, pl.ds(16 * d, 16)]
                    for d in range(NVEC))
                return new
            zeros = tuple(jnp.zeros((16,), jnp.float32) for _ in range(NVEC))
            accv = lax.fori_loop(0, MAXNN, mac, zeros)
            nl = c * (ECHUNK // MAXNN) + n4
            for d in range(NVEC):
                acc[nl, pl.ds(16 * d, 16)] = accv[d]

    for b in range(NBUF):
        issue(b, b)
    def round_body(r, _):
        for b in range(NBUF):
            c = r * NBUF + b
            @pl.when(c < NCHUNKS)
            def _():
                wait(c, b)
                compute(c, b)
                issue(c + NBUF, b)
        return 0
    lax.fori_loop(0, (NCHUNKS + NBUF - 1) // NBUF, round_body, 0)

    pltpu.sync_copy(acc, out_hbm.at[pl.ds(wid * NODES_PW, NODES_PW)])


@functools.partial(
    pl.kernel,
    out_type=jax.ShapeDtypeStruct((N, H), jnp.float32),
    mesh=plsc.VectorSubcoreMesh(core_axis_name="c", subcore_axis_name="s"),
    scratch_types=[
        pltpu.VMEM((NCHUNKS, ECHUNK), jnp.int32),
    ] + [pltpu.VMEM((ECHUNK, H), jnp.float32)] * 6
      + [pltpu.VMEM((NODES_PW, H), jnp.float32)]
      + [pltpu.SemaphoreType.DMA] * 6,
)
def _sc_aggregate(nbr_hbm, xs_hbm, wf_hbm, out_hbm, idx_v,
                  g0, g1, g2, w0, w1, w2, acc,
                  sg0, sg1, sg2, sw0, sw1, sw2):
    _sc_agg_body(nbr_hbm, xs_hbm, wf_hbm, out_hbm, idx_v,
                 g0, g1, g2, w0, w1, w2, acc,
                 sg0, sg1, sg2, sw0, sw1, sw2)


SEL_GROUP = 4

def _lanes():
    return lax.iota(jnp.int32, 16)


def _lane_perm(x, perm):
    dnums = jax.lax.GatherDimensionNumbers(
        offset_dims=(), collapsed_slice_dims=(0,), start_index_map=(0,))
    return jax.lax.gather(
        x, perm[:, None], dnums, (1,),
        mode=jax.lax.GatherScatterMode.PROMISE_IN_BOUNDS)


def _cmpex(key, val, j, cond_min_i):
    pk = _lane_perm(key, _lanes() ^ j)
    pv = _lane_perm(val, _lanes() ^ j)
    il = (_lanes() & j) == 0
    a_low = jnp.where(il, key, pk)
    a_high = jnp.where(il, pk, key)
    v_low = jnp.where(il, val, pv)
    v_high = jnp.where(il, pv, val)
    ci = jnp.where(a_low <= a_high, 1, 0)
    ce = (ci ^ cond_min_i) == 0
    return jnp.where(ce, a_low, a_high), jnp.where(ce, v_low, v_high)


def _sort16_desc(key, val):
    for k in (2, 4, 8, 16):
        sk = k.bit_length() - 1
        j = k // 2
        while j >= 1:
            sj = j.bit_length() - 1
            il_i = ((_lanes() >> sj) & 1) ^ 1
            asc_i = (_lanes() >> sk) & 1
            cond_min_i = (il_i ^ asc_i) ^ 1
            key, val = _cmpex(key, val, j, cond_min_i)
            j //= 2
    return key, val


def _bitonic_merge16_asc(key, val):
    for j in (8, 4, 2, 1):
        sj = j.bit_length() - 1
        cond_min_i = ((_lanes() >> sj) & 1) ^ 1
        key, val = _cmpex(key, val, j, cond_min_i)
    return key, val


def _merge32(a0, a1, i0, i1, bd, bi):
    c = a1 <= bd
    l1 = jnp.where(c, a1, bd)
    li1 = jnp.where(c, i1, bi)
    c2 = a0 <= l1
    m0 = jnp.where(c2, a0, l1)
    mi0 = jnp.where(c2, i0, li1)
    m1 = jnp.where(c2, l1, a0)
    mi1 = jnp.where(c2, li1, i0)
    a0, i0 = _bitonic_merge16_asc(m0, mi0)
    a1, i1 = _bitonic_merge16_asc(m1, mi1)
    return a0, a1, i0, i1


def _bf16r(x):
    c = x * 65537.0
    return c - (c - x)


def _bfly_min(x):
    for d in (1, 2, 4, 8):
        x = jnp.minimum(x, _lane_perm(x, _lanes() ^ d))
    return x


def _sc_select_body(posx_hbm, posy_hbm, posz_hbm, batch_hbm, seg_hbm,
                    idn_hbm, keys_hbm, nbr_hbm,
                    px_v, py_v, pz_v, bat_v, seg_v, idn_v, kbuf, ibuf):
    wid = lax.axis_index("s") * 2 + lax.axis_index("c")
    row0 = wid * NODES_PW
    pltpu.sync_copy(posx_hbm, px_v)
    pltpu.sync_copy(posy_hbm, py_v)
    pltpu.sync_copy(posz_hbm, pz_v)
    pltpu.sync_copy(batch_hbm, bat_v)
    pltpu.sync_copy(seg_hbm, seg_v)
    pltpu.sync_copy(idn_hbm, idn_v)

    zero16 = _lanes() * 0
    zp = zero16
    op = zero16 + 1
    inf16 = zero16.astype(jnp.float32) + jnp.inf

    def bcast0(x):
        return _lane_perm(x, zp)

    def group(k, _):
        rx, ry, rz, rsq, lov, hiv, rid = [], [], [], [], [], [], []
        ulov, uhiv = None, None
        for i in range(SEL_GROUP):
            r = row0 + SEL_GROUP * k + i
            xv = bcast0(px_v[pl.ds(r, 16)])
            yv = bcast0(py_v[pl.ds(r, 16)])
            zv = bcast0(pz_v[pl.ds(r, 16)])
            gv = bat_v[pl.ds(r, 16)]
            g0 = gv[0]
            sv = seg_v[pl.ds(g0, 16)]
            lo_s = sv[0]
            hi_s = sv[1]
            lov.append(bcast0(sv))
            hiv.append(_lane_perm(sv, op))
            ulov = lo_s if ulov is None else jnp.minimum(ulov, lo_s)
            uhiv = hi_s if uhiv is None else jnp.maximum(uhiv, hi_s)
            rx.append(_bf16r(xv))
            ry.append(_bf16r(yv))
            rz.append(_bf16r(zv))
            rsq.append(xv * xv + yv * yv + zv * zv)
            rid.append(bcast0(idn_v[pl.ds(r, 16)]))
        clo = ulov // 16
        chi = (uhiv + 15) // 16

        def chunk(c, carry):
            base = c * 16
            cx = px_v[pl.ds(base, 16)]
            cy = py_v[pl.ds(base, 16)]
            cz = pz_v[pl.ds(base, 16)]
            csq = cx * cx + cy * cy + cz * cz
            cxb = _bf16r(cx)
            cyb = _bf16r(cy)
            czb = _bf16r(cz)
            idxv = idn_v[pl.ds(base, 16)]
            for i in range(SEL_GROUP):
                rl = SEL_GROUP * k + i
                d2 = rsq[i] + csq - 2.0 * (rx[i] * cxb + ry[i] * cyb
                                           + rz[i] * czb)
                d2 = jnp.maximum(d2, 0.0)
                key = jnp.where(d2 > 0.0, d2, 1.0)
                bad = ((idxv < lov[i]) | (idxv >= hiv[i]) | (idxv == rid[i])
                       | (key >= CUTOFF * CUTOFF))
                key = jnp.where(bad, inf16, key)
                cmin = _bfly_min(key)
                a1pre = kbuf[rl, pl.ds(16, 16)]

                @pl.when(cmin[0] < a1pre[15])
                def _():
                    a0 = kbuf[rl, pl.ds(0, 16)]
                    i0 = ibuf[rl, pl.ds(0, 16)]
                    i1 = ibuf[rl, pl.ds(16, 16)]
                    bd, bi = _sort16_desc(key, idxv)
                    na0, na1, ni0, ni1 = _merge32(a0, a1pre, i0, i1, bd, bi)
                    kbuf[rl, pl.ds(0, 16)] = na0
                    kbuf[rl, pl.ds(16, 16)] = na1
                    ibuf[rl, pl.ds(0, 16)] = ni0
                    ibuf[rl, pl.ds(16, 16)] = ni1
            return 0

        for i in range(SEL_GROUP):
            rl = SEL_GROUP * k + i
            kbuf[rl, pl.ds(0, 16)] = inf16
            kbuf[rl, pl.ds(16, 16)] = inf16
            ibuf[rl, pl.ds(0, 16)] = zero16
            ibuf[rl, pl.ds(16, 16)] = zero16
        lax.fori_loop(clo, chi, chunk, 0)
        return 0

    lax.fori_loop(0, NODES_PW // SEL_GROUP, group, 0)
    pltpu.sync_copy(kbuf, keys_hbm.at[pl.ds(row0, NODES_PW)])
    pltpu.sync_copy(ibuf, nbr_hbm.at[pl.ds(row0, NODES_PW)])


@functools.partial(
    pl.kernel,
    out_type=(jax.ShapeDtypeStruct((N, MAXNN), jnp.float32),
              jax.ShapeDtypeStruct((N, MAXNN), jnp.int32)),
    mesh=plsc.VectorSubcoreMesh(core_axis_name="c", subcore_axis_name="s"),
    scratch_types=[
        pltpu.VMEM((N + 16,), jnp.float32),
        pltpu.VMEM((N + 16,), jnp.float32),
        pltpu.VMEM((N + 16,), jnp.float32),
        pltpu.VMEM((N + 16,), jnp.int32),
        pltpu.VMEM((32,), jnp.int32),
        pltpu.VMEM((N + 16,), jnp.int32),
        pltpu.VMEM((NODES_PW, MAXNN), jnp.float32),
        pltpu.VMEM((NODES_PW, MAXNN), jnp.int32),
    ],
)
def _sc_select(posx_hbm, posy_hbm, posz_hbm, batch_hbm, seg_hbm, idn_hbm,
               keys_hbm, nbr_hbm,
               px_v, py_v, pz_v, bat_v, seg_v, idn_v, kbuf, ibuf):
    _sc_select_body(posx_hbm, posy_hbm, posz_hbm, batch_hbm, seg_hbm,
                    idn_hbm, keys_hbm, nbr_hbm,
                    px_v, py_v, pz_v, bat_v, seg_v, idn_v, kbuf, ibuf)


def kernel(x, pos, batch, input_proj_w, input_proj_b, mlp_w1, mlp_b1,
           mlp_w2, mlp_b2, cf_lin1_w, cf_lin2_w, cf_lin2_b, int_lin_w,
           int_lin_b, out1_w, out1_b, out2_w, out2_b):
    h = _input_proj(x, input_proj_w.T, input_proj_b[None, :])

    batch_i = batch.astype(jnp.int32)
    pad_f = jnp.zeros((16,), jnp.float32)
    pad_i = jnp.zeros((16,), jnp.int32)
    seg = jnp.searchsorted(batch_i, jnp.arange(NGRAPHS + 1,
                                               dtype=jnp.int32)).astype(jnp.int32)
    seg32 = jnp.concatenate([seg, jnp.full((32 - NGRAPHS - 1,), N, jnp.int32)])
    idn = jnp.arange(N + 16, dtype=jnp.int32)
    keys, nbr = _sc_select(
        jnp.concatenate([pos[:, 0], pad_f]),
        jnp.concatenate([pos[:, 1], pad_f]),
        jnp.concatenate([pos[:, 2], pad_f]),
        jnp.concatenate([batch_i, pad_i]), seg32, idn)
    valid = jnp.isfinite(keys)
    w = jnp.where(valid, jnp.sqrt(keys), 0.0)
    ew = w.reshape(-1)
    vm = valid.reshape(-1).astype(jnp.float32)
    nbr_dma = nbr.reshape(N * MAXNN // ECHUNK, ECHUNK)

    offset = jnp.linspace(0.0, CUTOFF, NG)
    coeff = -0.5 / float(CUTOFF / (NG - 1)) ** 2
    edge_attr = jnp.exp(coeff * (ew[:, None] - offset[None, :]) ** 2)
    C = 0.5 * (jnp.cos(ew * jnp.pi / CUTOFF) + 1.0) * vm

    for l in range(L):
        Wf = _ssp(edge_attr @ mlp_w1[l].T + mlp_b1[l]) @ mlp_w2[l].T + mlp_b2[l]
        Wf = Wf * C[:, None]
        xs = h @ cf_lin1_w[l].T
        agg = _sc_aggregate(nbr_dma, xs, Wf)
        hc = _ssp(agg @ cf_lin2_w[l].T + cf_lin2_b[l])
        hc = hc @ int_lin_w[l].T + int_lin_b[l]
        h = h + hc

    h = _ssp(h @ out1_w.T + out1_b)
    h = h @ out2_w.T + out2_b
    return jax.ops.segment_sum(h, batch, num_segments=NGRAPHS)
